# Initial kernel scaffold; baseline (speedup 1.0000x reference)
import jax
import jax.numpy as jnp
from jax.experimental import pallas as pl


def kernel(pointcloud, W1, W2, W3, W4):
    raise NotImplementedError("write your pallas kernel here")


# speedup vs baseline: 5.0628x; 5.0628x over previous
import functools

import jax
import jax.numpy as jnp
from jax import lax
from jax.experimental import pallas as pl
from jax.experimental.pallas import tpu as pltpu
from jax.experimental.pallas import tpu_sc as plsc

_B = 4
_N = 20000
_NPTS = 1024
_ROWS = 160
_NPAD = _ROWS * 128
_VOXEL = 1e-06
_KMAX = 1 << 20
_M = _B * _NPTS
_NK = 27
_QL = _NK * _M
_NL = 4
_NQ = _NL * _QL
_NW = 32
_QPT = _NQ // _NW
_GPT = _QL // _NW
_CH = 128
_NCH = _GPT // _CH

_OFFS = [(dx, dy, dz) for dx in (-1, 0, 1) for dy in (-1, 0, 1) for dz in (-1, 0, 1)]


def _fps_body(coords_ref, out_ref):
    x = coords_ref[0]
    y = coords_ref[1]
    z = coords_ref[2]
    shp = (_B, _ROWS, 128)
    rid = lax.broadcasted_iota(jnp.int32, shp, 1)
    cid = lax.broadcasted_iota(jnp.int32, shp, 2)
    pid = rid * 128 + cid
    pad = pid >= _N
    islot = (lax.broadcasted_iota(jnp.int32, (_B, 8, 128), 1) * 128
             + lax.broadcasted_iota(jnp.int32, (_B, 8, 128), 2))

    lane = lax.broadcasted_iota(jnp.int32, (1, 128), 1)
    big = jnp.full(shp, _NPAD, jnp.int32)

    def dist_update(dists, nxt):
        sxs, sys_, szs = [], [], []
        for b in range(_B):
            nb = nxt[b, 0, 0]
            r = nb // jnp.int32(128)
            cc = nb % jnp.int32(128)
            xrow = coords_ref[0, b, pl.ds(r, 1), :]
            yrow = coords_ref[1, b, pl.ds(r, 1), :]
            zrow = coords_ref[2, b, pl.ds(r, 1), :]
            sel = lane == cc
            sxs.append(jnp.sum(jnp.where(sel, xrow, 0.0)))
            sys_.append(jnp.sum(jnp.where(sel, yrow, 0.0)))
            szs.append(jnp.sum(jnp.where(sel, zrow, 0.0)))
        sx = jnp.stack(sxs).reshape(_B, 1, 1)
        sy = jnp.stack(sys_).reshape(_B, 1, 1)
        sz = jnp.stack(szs).reshape(_B, 1, 1)
        dx = x - sx
        dy = y - sy
        dz = z - sz
        d = (dx * dx + dz * dz) + dy * dy
        return jnp.minimum(dists, d)

    zero = jnp.zeros((_B, 1, 1), jnp.int32)
    inf0 = jnp.where(pad, -jnp.inf, jnp.float32(jnp.inf))
    dists = dist_update(inf0, zero)
    ibuf = jnp.zeros((_B, 8, 128), jnp.int32)

    def body(i, carry):
        dists, ibuf = carry
        m = jnp.max(dists, axis=(1, 2), keepdims=True)
        nxt = jnp.min(jnp.where(dists == m, pid, big),
                      axis=(1, 2), keepdims=True)
        ibuf = jnp.where(islot == i, nxt, ibuf)
        dists = dist_update(dists, nxt)
        return dists, ibuf

    _, ibuf = lax.fori_loop(jnp.int32(1), jnp.int32(_NPTS), body,
                            (dists, ibuf))
    out_ref[...] = ibuf


def _fps(coords3):
    cpad = jnp.pad(coords3, ((0, 0), (0, _NPAD - _N), (0, 0)))
    cpl = jnp.transpose(cpad, (2, 0, 1)).reshape(3, _B, _ROWS, 128)
    ibuf = pl.pallas_call(
        _fps_body,
        out_shape=jax.ShapeDtypeStruct((_B, 8, 128), jnp.int32),
    )(cpl)
    return ibuf.reshape(_B, _NPTS)


def _keys(c):
    hi = (c[..., 0] << 10) | (c[..., 1] >> 10)
    lo = ((c[..., 1] & 1023) << 20) | c[..., 2]
    return hi, lo


def _layer_tables(c):
    hi, lo = _keys(c)
    order = jnp.lexsort((lo, hi))
    skhi = jnp.take_along_axis(hi, order, 1).reshape(-1)
    sklo = jnp.take_along_axis(lo, order, 1).reshape(-1)
    ordg = (order.astype(jnp.int32)
            + (jnp.arange(_B, dtype=jnp.int32) * _NPTS)[:, None]).reshape(-1)
    return skhi, sklo, ordg


def _layer_queries(c):
    base = (c >> 1) * 2
    offs = jnp.array(_OFFS, dtype=jnp.int32)
    q = base[None] + offs[:, None, None, :]
    valid = jnp.all((q >= 0) & (q < _KMAX), axis=-1)
    qc = jnp.maximum(q, 0)
    qhi, qlo = _keys(qc)
    qhi = jnp.where(valid, qhi, jnp.int32(0x7FFFFFFF))
    return qhi.reshape(-1), qlo.reshape(-1)


def _sc_index(skhi, sklo, ordg, qhi, qlo):
    mesh = plsc.VectorSubcoreMesh(core_axis_name="c", subcore_axis_name="s")

    @functools.partial(
        pl.kernel, mesh=mesh,
        out_type=jax.ShapeDtypeStruct((_NQ,), jnp.int32),
        scratch_types=[
            pltpu.VMEM((_M,), jnp.int32),
            pltpu.VMEM((_M,), jnp.int32),
            pltpu.VMEM((_M,), jnp.int32),
            pltpu.VMEM((_QPT,), jnp.int32),
            pltpu.VMEM((_QPT,), jnp.int32),
            pltpu.VMEM((_QPT,), jnp.int32),
        ],
        compiler_params=pltpu.CompilerParams(needs_layout_passes=False),
    )
    def k(skhi_h, sklo_h, ordg_h, qhi_h, qlo_h, out_h,
          skv, slv, ogv, qhv, qlv, gv):
        wid = lax.axis_index("s") * 2 + lax.axis_index("c")
        lyr = wid // 8
        qbase = pl.multiple_of(wid * _QPT, 128)
        pltpu.sync_copy(skhi_h.at[lyr], skv)
        pltpu.sync_copy(sklo_h.at[lyr], slv)
        pltpu.sync_copy(ordg_h.at[lyr], ogv)
        pltpu.sync_copy(qhi_h.at[pl.ds(qbase, _QPT)], qhv)
        pltpu.sync_copy(qlo_h.at[pl.ds(qbase, _QPT)], qlv)
        lane = lax.broadcasted_iota(jnp.int32, (16,), 0)
        qrem = jnp.full((16,), 1, jnp.int32) * lax.rem(qbase, jnp.int32(_M))

        @plsc.parallel_loop(jnp.int32(0), jnp.int32(_QPT // 16),
                            jnp.int32(1), unroll=2)
        def step(j):
            jj = pl.multiple_of(j * 16, 16)
            qh = qhv[pl.ds(jj, 16)]
            ql = qlv[pl.ds(jj, 16)]
            bi = (qrem + j * 16 + lane) & jnp.int32(_M - 1)
            seg = bi & jnp.int32(~(_NPTS - 1))
            pos = seg
            for s in (512, 256, 128, 64, 32, 16, 8, 4, 2, 1):
                probe = pos + jnp.int32(s - 1)
                h = plsc.load_gather(skv, [probe])
                l2 = plsc.load_gather(slv, [probe])
                less = (h < qh) | ((h == qh) & (l2 < ql))
                pos = jnp.where(less, pos + jnp.int32(s), pos)
            posc = jnp.minimum(pos, seg + jnp.int32(_NPTS - 1))
            h = plsc.load_gather(skv, [posc])
            l2 = plsc.load_gather(slv, [posc])
            hit = (pos < seg + jnp.int32(_NPTS)) & (h == qh) & (l2 == ql)
            src = plsc.load_gather(ogv, [posc])
            row = jnp.where(hit, src, jnp.int32(_M))
            gv[pl.ds(jj, 16)] = row

        pltpu.sync_copy(gv, out_h.at[pl.ds(qbase, _QPT)])

    return k(skhi, sklo, ordg, qhi, qlo)


def _sc_gather(table, gidx, cin):
    mesh = plsc.VectorSubcoreMesh(core_axis_name="c", subcore_axis_name="s")

    @functools.partial(
        pl.kernel, mesh=mesh,
        out_type=jax.ShapeDtypeStruct((_QL, cin), jnp.float32),
        scratch_types=[
            pltpu.VMEM((_GPT,), jnp.int32),
            pltpu.VMEM((_CH, cin), jnp.float32),
            pltpu.VMEM((_CH, cin), jnp.float32),
            pltpu.SemaphoreType.DMA,
            pltpu.SemaphoreType.DMA,
        ],
        compiler_params=pltpu.CompilerParams(use_tc_tiling_on_sc=False),
    )
    def k(tab_h, idx_h, out_h, idxv, buf0, buf1, sem0, sem1):
        wid = lax.axis_index("s") * 2 + lax.axis_index("c")
        rbase = pl.multiple_of(wid * _GPT, 128)
        pltpu.sync_copy(idx_h.at[pl.ds(rbase, _GPT)], idxv)
        bufs = (buf0, buf1)
        sems = (sem0, sem1)
        cps = [pltpu.async_copy(tab_h.at[idxv.at[pl.ds(jnp.int32(0), _CH)]], buf0, sem0)]

        for c in range(_NCH):
            if c + 1 < _NCH:
                cps.append(pltpu.async_copy(
                    tab_h.at[idxv.at[pl.ds(jnp.int32((c + 1) * _CH), _CH)]], bufs[(c + 1) % 2], sems[(c + 1) % 2]))
            cps[c].wait()
            pltpu.sync_copy(bufs[c % 2], out_h.at[pl.ds(rbase + c * _CH, _CH)])

    return k(table, gidx)


def _mm_body(g_ref, w_ref, o_ref):
    @pl.when(pl.program_id(0) == 0)
    def _():
        o_ref[...] = jnp.zeros_like(o_ref)
    o_ref[...] += jnp.dot(g_ref[0], w_ref[0],
                          preferred_element_type=jnp.float32)


def _mm(g, w):
    cin, cout = w.shape[1], w.shape[2]
    return pl.pallas_call(
        _mm_body,
        grid=(_NK,),
        in_specs=[
            pl.BlockSpec((1, _M, cin), lambda k: (k, jnp.int32(0), jnp.int32(0))),
            pl.BlockSpec((1, cin, cout), lambda k: (k, jnp.int32(0), jnp.int32(0))),
        ],
        out_specs=pl.BlockSpec((_M, cout),
                               lambda k: (jnp.int32(0), jnp.int32(0))),
        out_shape=jax.ShapeDtypeStruct((_M, cout), jnp.float32),
    )(g, w)


def _conv_chain(sp, feats, Ws):
    skhis, sklos, ordgs, qhis, qlos = [], [], [], [], []
    c = sp
    for _ in range(_NL):
        skhi, sklo, ordg = _layer_tables(c)
        qhi, qlo = _layer_queries(c)
        skhis.append(skhi)
        sklos.append(sklo)
        ordgs.append(ordg)
        qhis.append(qhi)
        qlos.append(qlo)
        c = c >> 1
    gidx = _sc_index(jnp.stack(skhis), jnp.stack(sklos), jnp.stack(ordgs),
                     jnp.concatenate(qhis), jnp.concatenate(qlos))
    f = feats
    for l in range(_NL):
        cin = max(f.shape[1], 8)
        fe = jnp.zeros((_M + 8, cin), jnp.float32)
        fe = fe.at[:_M, :f.shape[1]].set(f)
        g2 = lax.dynamic_slice_in_dim(gidx, l * _QL, _QL)
        g = _sc_gather(fe, g2, cin).reshape(_NK, _M, cin)
        w = Ws[l].astype(jnp.float32)
        if cin != w.shape[1]:
            w = jnp.pad(w, ((0, 0), (0, cin - w.shape[1]), (0, 0)))
        f = _mm(g, w)
    return f


def kernel(pointcloud, W1, W2, W3, W4):
    B, N, C = pointcloud.shape
    coords3 = pointcloud[:, :, :3]
    idxs = _fps(lax.stop_gradient(coords3))
    flat_idx = (idxs + (jnp.arange(B, dtype=jnp.int32) * N)[:, None]).reshape(-1)
    sampled_coords = coords3.reshape(-1, 3)[flat_idx]
    sampled_feats = pointcloud[:, :, 3:].reshape(-1, C - 3)[flat_idx]
    sp = jnp.floor(lax.stop_gradient(sampled_coords) / _VOXEL).astype(jnp.int32)
    f = _conv_chain(sp.reshape(B, _NPTS, 3),
                    sampled_feats.astype(jnp.float32), [W1, W2, W3, W4])
    features = jnp.transpose(f.reshape(B, _NPTS, 256), (0, 2, 1)).astype(W1.dtype)
    xyz = sampled_coords.reshape(B, _NPTS, 3)
    inds = jnp.tile(jnp.arange(_NPTS, dtype=jnp.int32)[None, :], (B, 1))
    return features, xyz, inds
